# baseline (speedup 1.0000x reference)
import functools
import jax
import jax.numpy as jnp
from jax import lax
from jax.experimental import pallas as pl
from jax.experimental.pallas import tpu as pltpu
from jax.experimental.pallas import tpu_sc as plsc

N = 10000
E = 320000
F = 128
C = 64
C2 = C // 2

NC = 2
NS = 16
NW = NC * NS

BATCH = 128
NCHUNK = E // BATCH
CPW = NCHUNK // NW
XTRA = NCHUNK % NW

N_PAD = 10240
RPW = N_PAD // NS

_mesh = plsc.VectorSubcoreMesh(core_axis_name="c", subcore_axis_name="s")


def _chunk_range(wid):
    extra = (wid < XTRA).astype(jnp.int32)
    off = wid * CPW + jnp.minimum(wid, XTRA)
    return off, CPW + extra


@functools.partial(
    pl.kernel,
    out_type=jax.ShapeDtypeStruct((NC, N_PAD), jnp.float32),
    mesh=_mesh,
    scratch_types=[
        pltpu.VMEM(((CPW + 1) * BATCH,), jnp.int32),
        pltpu.VMEM((BATCH,), jnp.float32),
        pltpu.VMEM((BATCH,), jnp.float32),
        pltpu.VMEM_SHARED((N_PAD,), jnp.float32),
    ],
    compiler_params=pltpu.CompilerParams(use_tc_tiling_on_sc=False),
)
def _deg_kernel(adj_hbm, degp_hbm, dst_v, ones_v, zero_v, deg_sh):
    c = lax.axis_index("c")
    s = lax.axis_index("s")
    wid = c * NS + s
    off, nch = _chunk_range(wid)

    one = jnp.ones((16,), jnp.float32)
    z = jnp.zeros((16,), jnp.float32)
    for j in range(BATCH // 16):
        ones_v[pl.ds(j * 16, 16)] = one
        zero_v[pl.ds(j * 16, 16)] = z

    base = s * RPW
    for k in range(RPW // BATCH):
        pltpu.sync_copy(zero_v, deg_sh.at[pl.ds(base + k * BATCH, BATCH)])

    eoff = off * BATCH

    @pl.when(wid < XTRA)
    def _():
        pltpu.sync_copy(adj_hbm.at[1, pl.ds(eoff, (CPW + 1) * BATCH)], dst_v)

    @pl.when(wid >= XTRA)
    def _():
        pltpu.sync_copy(adj_hbm.at[1, pl.ds(eoff, CPW * BATCH)],
                        dst_v.at[pl.ds(0, CPW * BATCH)])

    plsc.subcore_barrier()

    @pl.loop(0, nch)
    def _(j):
        pltpu.sync_copy(ones_v, deg_sh.at[dst_v.at[pl.ds(j * BATCH, BATCH)]], add=True)

    plsc.subcore_barrier()
    pltpu.sync_copy(deg_sh.at[pl.ds(base, RPW)],
                    degp_hbm.at[c, pl.ds(base, RPW)])


NBUF = 2


@functools.partial(
    pl.kernel,
    out_type=jax.ShapeDtypeStruct((NC, N_PAD, 128), jnp.float32),
    mesh=_mesh,
    scratch_types=[
        pltpu.VMEM(((CPW + 1) * BATCH,), jnp.int32),
        pltpu.VMEM(((CPW + 1) * BATCH,), jnp.int32),
        [pltpu.VMEM((BATCH, C2), jnp.float32)] * NBUF,
        pltpu.VMEM((BATCH, C2), jnp.float32),
        [pltpu.VMEM_SHARED((N_PAD, C2), jnp.float32)] * 2,
        pltpu.VMEM_SHARED((N_PAD, C2), jnp.float32),
        [pltpu.SemaphoreType.DMA] * NBUF,
        [pltpu.SemaphoreType.DMA] * NBUF,
        pltpu.SemaphoreType.DMA,
    ],
    compiler_params=pltpu.CompilerParams(use_tc_tiling_on_sc=False),
)
def _agg_kernel(h2s_hbm, adj_hbm, accp_hbm,
                src_v, dst_v, bufs, zbuf, h2_shs, acc_sh, gsems, ssems, psem):
    c = lax.axis_index("c")
    s = lax.axis_index("s")
    wid = c * NS + s
    off, nch = _chunk_range(wid)
    base = s * RPW

    z = jnp.zeros((16,), jnp.float32)

    @pl.loop(0, BATCH)
    def _(i):
        for j in range(C2 // 16):
            zbuf[i, pl.ds(j * 16, 16)] = z

    eoff = off * BATCH

    @pl.when(wid < XTRA)
    def _():
        pltpu.sync_copy(adj_hbm.at[0, pl.ds(eoff, (CPW + 1) * BATCH)], src_v)
        pltpu.sync_copy(adj_hbm.at[1, pl.ds(eoff, (CPW + 1) * BATCH)], dst_v)

    @pl.when(wid >= XTRA)
    def _():
        pltpu.sync_copy(adj_hbm.at[0, pl.ds(eoff, CPW * BATCH)],
                        src_v.at[pl.ds(0, CPW * BATCH)])
        pltpu.sync_copy(adj_hbm.at[1, pl.ds(eoff, CPW * BATCH)],
                        dst_v.at[pl.ds(0, CPW * BATCH)])

    pltpu.sync_copy(h2s_hbm.at[pl.ds(base, RPW), pl.ds(0, C2)],
                    h2_shs[0].at[pl.ds(base, RPW)])
    pltpu.async_copy(h2s_hbm.at[pl.ds(base, RPW), pl.ds(C2, C2)],
                     h2_shs[1].at[pl.ds(base, RPW)], psem)

    for p in range(2):
        h2_sh = h2_shs[p]
        if p == 1:
            pltpu.make_async_copy(h2s_hbm.at[pl.ds(base, RPW), pl.ds(C2, C2)],
                                  h2_shs[1].at[pl.ds(base, RPW)], psem).wait()
        for k in range(RPW // BATCH):
            pltpu.sync_copy(zbuf, acc_sh.at[pl.ds(base + k * BATCH, BATCH)])
        plsc.subcore_barrier()

        for b in range(NBUF):
            pltpu.async_copy(h2_sh.at[src_v.at[pl.ds(b * BATCH, BATCH)]], bufs[b], gsems[b])

        @pl.loop(0, CPW, step=NBUF)
        def _(j):
            for b in range(NBUF):
                jj = j + b
                pltpu.make_async_copy(h2_sh.at[src_v.at[pl.ds(jj * BATCH, BATCH)]], bufs[b], gsems[b]).wait()
                pltpu.async_copy(bufs[b], acc_sh.at[dst_v.at[pl.ds(jj * BATCH, BATCH)]], ssems[b], add=True)
            for b in range(NBUF):
                nxt = j + b + NBUF

                @pl.when(nxt < CPW)
                def _():
                    pltpu.make_async_copy(
                        bufs[b], acc_sh.at[dst_v.at[pl.ds((nxt - NBUF) * BATCH, BATCH)]], ssems[b]).wait()
                    pltpu.async_copy(h2_sh.at[src_v.at[pl.ds(nxt * BATCH, BATCH)]], bufs[b], gsems[b])

        for b in range(NBUF):
            pltpu.make_async_copy(
                bufs[b], acc_sh.at[dst_v.at[pl.ds((CPW - NBUF + b) * BATCH, BATCH)]], ssems[b]).wait()

        @pl.when(nch > CPW)
        def _():
            pltpu.sync_copy(h2_sh.at[src_v.at[pl.ds(CPW * BATCH, BATCH)]], zbuf)
            pltpu.sync_copy(zbuf, acc_sh.at[dst_v.at[pl.ds(CPW * BATCH, BATCH)]], add=True)

        plsc.subcore_barrier()
        pltpu.sync_copy(acc_sh.at[pl.ds(base, RPW)],
                        accp_hbm.at[c, pl.ds(base, RPW), pl.ds(p * C2, C2)])

        if p == 0:
            @pl.when(nch > CPW)
            def _():
                @pl.loop(0, BATCH)
                def _(i):
                    for j in range(C2 // 16):
                        zbuf[i, pl.ds(j * 16, 16)] = z


RB = 1024


def _mm_body(x_ref, w_ref, h_ref):
    h = jnp.dot(x_ref[...], w_ref[...], preferred_element_type=jnp.float32)
    h_ref[...] = jnp.concatenate([h, h], axis=1)


def _scale_body(deg_ref, h_ref, h2f_ref):
    deg = deg_ref[0] + deg_ref[1] + 1.0
    dinv = lax.rsqrt(deg)
    h2f_ref[...] = h_ref[...] * dinv[:, None]


def _fin_body(deg_ref, acc_ref, h2f_ref, b_ref, out_ref):
    deg = deg_ref[0] + deg_ref[1] + 1.0
    dinv = lax.rsqrt(deg)
    tot = acc_ref[0, :, :C] + acc_ref[1, :, :C] + h2f_ref[:, :C]
    out_ref[...] = tot * dinv[:, None] + b_ref[...]


def _tc_mm(x, W):
    return pl.pallas_call(
        _mm_body,
        grid=(N_PAD // RB,),
        in_specs=[
            pl.BlockSpec((RB, F), lambda i: (i, 0)),
            pl.BlockSpec((F, C), lambda i: (0, 0)),
        ],
        out_specs=pl.BlockSpec((RB, 128), lambda i: (i, 0)),
        out_shape=jax.ShapeDtypeStruct((N_PAD, 128), jnp.float32),
    )(x, W)


def _tc_scale(degp, h):
    return pl.pallas_call(
        _scale_body,
        grid=(N_PAD // RB,),
        in_specs=[
            pl.BlockSpec((NC, RB), lambda i: (0, i)),
            pl.BlockSpec((RB, 128), lambda i: (i, 0)),
        ],
        out_specs=pl.BlockSpec((RB, 128), lambda i: (i, 0)),
        out_shape=jax.ShapeDtypeStruct((N_PAD, 128), jnp.float32),
    )(degp, h)


def _tc_final(degp, accp, h2s, b):
    return pl.pallas_call(
        _fin_body,
        grid=(N_PAD // RB,),
        in_specs=[
            pl.BlockSpec((NC, RB), lambda i: (0, i)),
            pl.BlockSpec((NC, RB, 128), lambda i: (0, i, 0)),
            pl.BlockSpec((RB, 128), lambda i: (i, 0)),
            pl.BlockSpec((1, C), lambda i: (0, 0)),
        ],
        out_specs=pl.BlockSpec((RB, C), lambda i: (i, 0)),
        out_shape=jax.ShapeDtypeStruct((N, C), jnp.float32),
    )(degp, accp, h2s, b)


def kernel(x, adj, W, b):
    adj2d = adj.astype(jnp.int32)

    degp = _deg_kernel(adj2d)
    h = _tc_mm(x, W)
    h2s = _tc_scale(degp, h)
    accp = _agg_kernel(h2s, adj2d)
    return _tc_final(degp, accp, h2s, b.reshape(1, C))
